# baseline (speedup 1.0000x reference)
import functools

import jax
import jax.numpy as jnp
from jax import lax
from jax.experimental import pallas as pl
from jax.experimental.pallas import tpu as pltpu
from jax.experimental.pallas import tpu_sc as plsc

VOCAB = 1000000
EMBED = 64
B = 4096 * 200
IDXW = 128
NW = 32
ROWS_PER_W = B // NW
IDX_ROWS_W = ROWS_PER_W // IDXW
CHUNK = 128
NCH = ROWS_PER_W // CHUNK
GPC = CHUNK // IDXW
SCALE = 8.0

_mesh = plsc.VectorSubcoreMesh(core_axis_name="c", subcore_axis_name="s")


@functools.partial(
    pl.kernel,
    mesh=_mesh,
    out_type=jax.ShapeDtypeStruct((B, EMBED), jnp.float32),
    scratch_types=[
        pltpu.VMEM((IDX_ROWS_W, IDXW), jnp.int32),
        [pltpu.VMEM((CHUNK, 2 * EMBED), jnp.float32) for _ in range(2)],
        [pltpu.VMEM((CHUNK, EMBED), jnp.float32) for _ in range(2)],
        [pltpu.SemaphoreType.DMA for _ in range(2)],
    ],
    compiler_params=pltpu.CompilerParams(use_tc_tiling_on_sc=True),
)
def _embed(idx_hbm, tbl_hbm, out_hbm, tok_v, rows_v, outf_v, sems):
    wid = lax.axis_index("s") * 2 + lax.axis_index("c")
    row_base = wid * ROWS_PER_W
    irow_base = pl.multiple_of(row_base // IDXW, IDX_ROWS_W)
    pltpu.sync_copy(idx_hbm.at[pl.ds(irow_base, IDX_ROWS_W)], tok_v)

    def fire(n, buf):
        for j in range(GPC):
            pltpu.async_copy(
                tbl_hbm.at[tok_v.at[n * GPC + j]],
                rows_v[buf].at[pl.ds(j * IDXW, IDXW)],
                sems[buf],
            )

    def drain(n, buf):
        for j in range(GPC):
            pltpu.make_async_copy(
                tbl_hbm.at[tok_v.at[n * GPC + j]],
                rows_v[buf].at[pl.ds(j * IDXW, IDXW)],
                sems[buf],
            ).wait()

    fire(0, 0)
    def group_body(g, carry):
        for b in range(2):
            n = g * 2 + b

            @pl.when(n + 1 < NCH)
            def _():
                fire(n + 1, 1 - b)

            drain(n, b)

            def row_body(i, c):
                for j in range(EMBED // 16):
                    outf_v[b][i, pl.ds(j * 16, 16)] = rows_v[b][
                        i, pl.ds(j * 16, 16)
                    ]
                return c

            lax.fori_loop(0, CHUNK, row_body, 0, unroll=4)
            pltpu.sync_copy(
                outf_v[b],
                out_hbm.at[pl.ds(row_base + n * CHUNK, CHUNK)],
            )
        return carry

    lax.fori_loop(0, NCH // 2, group_body, 0)


def kernel(token_ids, table):
    ids = token_ids.astype(jnp.int32).reshape(B // IDXW, IDXW)
    scaled = table * SCALE
    shifted = jnp.concatenate([scaled[1:], scaled[:1]], axis=0)
    tbl_wide = jnp.concatenate([scaled, shifted], axis=1)
    out = _embed(ids, tbl_wide)
    return out.reshape(token_ids.shape + (EMBED,))


# speedup vs baseline: 1.2154x; 1.2154x over previous
import functools

import jax
import jax.numpy as jnp
from jax import lax
from jax.experimental import pallas as pl
from jax.experimental.pallas import tpu as pltpu
from jax.experimental.pallas import tpu_sc as plsc

VOCAB = 1000000
EMBED = 64
B = 4096 * 200
IDXW = 128
NW = 32
ROWS_PER_W = B // NW
IDX_ROWS_W = ROWS_PER_W // IDXW
CHUNK = 256
NCH = ROWS_PER_W // CHUNK
GPC = CHUNK // IDXW
SCALE = 8.0

_mesh = plsc.VectorSubcoreMesh(core_axis_name="c", subcore_axis_name="s")


@functools.partial(
    pl.kernel,
    mesh=_mesh,
    out_type=jax.ShapeDtypeStruct((B // 2, 2 * EMBED), jnp.float32),
    scratch_types=[
        pltpu.VMEM((IDX_ROWS_W, IDXW), jnp.int32),
        [pltpu.VMEM((CHUNK, EMBED), jnp.float32) for _ in range(2)],
        [pltpu.VMEM((CHUNK // 2, 2 * EMBED), jnp.float32) for _ in range(2)],
        [pltpu.SemaphoreType.DMA for _ in range(2)],
    ],
    compiler_params=pltpu.CompilerParams(use_tc_tiling_on_sc=False),
)
def _embed(idx_hbm, tbl_hbm, out_hbm, tok_v, rows_v, outw_v, sems):
    wid = lax.axis_index("s") * 2 + lax.axis_index("c")
    row_base = wid * ROWS_PER_W
    irow_base = pl.multiple_of(row_base // IDXW, IDX_ROWS_W)
    pltpu.sync_copy(idx_hbm.at[pl.ds(irow_base, IDX_ROWS_W)], tok_v)

    def fire(n, buf):
        for j in range(GPC):
            pltpu.async_copy(
                tbl_hbm.at[tok_v.at[n * GPC + j]],
                rows_v[buf].at[pl.ds(j * IDXW, IDXW)],
                sems[buf],
            )

    def drain(n, buf):
        for j in range(GPC):
            pltpu.make_async_copy(
                tbl_hbm.at[tok_v.at[n * GPC + j]],
                rows_v[buf].at[pl.ds(j * IDXW, IDXW)],
                sems[buf],
            ).wait()

    fire(0, 0)
    def group_body(g, carry):
        for b in range(2):
            n = g * 2 + b

            @pl.when(n + 1 < NCH)
            def _():
                fire(n + 1, 1 - b)

            drain(n, b)

            def pack_body(i2, c):
                for r in range(2):
                    for j in range(EMBED // 16):
                        outw_v[b][i2, pl.ds(r * EMBED + j * 16, 16)] = (
                            rows_v[b][2 * i2 + r, pl.ds(j * 16, 16)] * SCALE
                        )
                return c

            lax.fori_loop(0, CHUNK // 2, pack_body, 0, unroll=4)
            pltpu.sync_copy(
                outw_v[b],
                out_hbm.at[pl.ds((row_base + n * CHUNK) // 2, CHUNK // 2)],
            )
        return carry

    lax.fori_loop(0, NCH // 2, group_body, 0)


def kernel(token_ids, table):
    ids = token_ids.astype(jnp.int32).reshape(B // IDXW, IDXW)
    out = _embed(ids, table)
    return out.reshape(token_ids.shape + (EMBED,))


# speedup vs baseline: 1.4845x; 1.2214x over previous
import functools

import jax
import jax.numpy as jnp
from jax import lax
from jax.experimental import pallas as pl
from jax.experimental.pallas import tpu as pltpu
from jax.experimental.pallas import tpu_sc as plsc

VOCAB = 1000000
EMBED = 64
B = 4096 * 200
IDXW = 128
NW = 32
ROWS_PER_W = B // NW
SUPER = 1024
CHUNK = 512
N_SUPER = ROWS_PER_W // SUPER
SCALE = 8.0

_mesh = plsc.VectorSubcoreMesh(core_axis_name="c", subcore_axis_name="s")


@functools.partial(
    pl.kernel,
    mesh=_mesh,
    out_type=jax.ShapeDtypeStruct((B, EMBED), jnp.float32),
    scratch_types=[
        pltpu.VMEM((SUPER // IDXW, IDXW), jnp.int32),
        pltpu.VMEM((CHUNK, EMBED), jnp.float32),
        pltpu.SemaphoreType.DMA,
    ],
    compiler_params=pltpu.CompilerParams(
        use_tc_tiling_on_sc=False, needs_layout_passes=False
    ),
)
def _embed(idx_hbm, table_hbm, out_hbm, idx_v, rows_v, sem):
    wid = lax.axis_index("s") * 2 + lax.axis_index("c")
    row_base = wid * ROWS_PER_W

    def super_body(si, carry):
        srow0 = pl.multiple_of(row_base + si * SUPER, SUPER)
        irow0 = pl.multiple_of(srow0 // IDXW, SUPER // IDXW)
        pltpu.sync_copy(idx_hbm.at[pl.ds(irow0, SUPER // IDXW)], idx_v)
        for h in range(SUPER // CHUNK):
            copies = []
            for j in range(CHUNK // IDXW):
                copies.append(
                    pltpu.async_copy(
                        table_hbm.at[idx_v.at[h * (CHUNK // IDXW) + j]],
                        rows_v.at[pl.ds(j * IDXW, IDXW)],
                        sem,
                    )
                )
            for c in copies:
                c.wait()

            def scale_body(i, c):
                for j in range(EMBED // 16):
                    sl = (i, pl.ds(j * 16, 16))
                    rows_v[sl] = rows_v[sl] * SCALE
                return c

            lax.fori_loop(0, CHUNK, scale_body, 0, unroll=2)
            pltpu.sync_copy(
                rows_v, out_hbm.at[pl.ds(srow0 + h * CHUNK, CHUNK)]
            )
        return carry

    lax.fori_loop(0, N_SUPER, super_body, 0)


def kernel(token_ids, table):
    ids = token_ids.astype(jnp.int32).reshape(B // IDXW, IDXW)
    out = _embed(ids, table)
    return out.reshape(token_ids.shape + (EMBED,))
